# baseline (speedup 1.0000x reference)
import functools

import jax
import jax.numpy as jnp
from jax import lax
from jax.experimental import pallas as pl
from jax.experimental.pallas import tpu as pltpu
from jax.experimental.pallas import tpu_sc as plsc

NUM_ROWS = 16384
NUM_COLS = 20
EMBED_DIM = 32
B = NUM_ROWS * NUM_COLS

NC = 2
NS = 16
NW = NC * NS
ROWS_PER_W = NUM_ROWS // NW
B_PER_W = B // NW
RBLK = 128
HALF = RBLK // 2
CHUNK = HALF * NUM_COLS
N_RBLK = ROWS_PER_W // RBLK

OUT5 = (NUM_COLS, EMBED_DIM // 8, NUM_ROWS // 128, 8, 128)

NUM_V = 1000000
TCOL = 128
N_TCOLS = NUM_V // TCOL
TAIL = NUM_V - N_TCOLS * TCOL
GRP = 8
N_GRP = N_TCOLS // GRP
GRP_PER_W = (N_GRP + 1 + NW - 1) // NW


@functools.partial(
    pl.kernel,
    mesh=plsc.VectorSubcoreMesh(core_axis_name="c", subcore_axis_name="s"),
    out_type=(
        jax.ShapeDtypeStruct((NUM_V // 4, 128), jnp.float32),
        jax.ShapeDtypeStruct((NUM_V // 4, 128), jnp.float32),
    ),
    scratch_types=[
        pltpu.VMEM((32, GRP * TCOL), jnp.float32),
        pltpu.VMEM((GRP * 32, 128), jnp.float32),
        pltpu.SemaphoreType.DMA,
    ],
    compiler_params=pltpu.CompilerParams(
        use_tc_tiling_on_sc=True, needs_layout_passes=False),
)
def _repack(wt_sr, wt_tg, tail_sr, tail_tg, t4_sr, t4_tg, in_v, out_b, wsem):
    wid = lax.axis_index("s") * NC + lax.axis_index("c")
    iota16 = lax.iota(jnp.int32, 16)

    def transpose_cols(n_cols, lane0_in, row0_out):
        @plsc.parallel_loop(0, n_cols * 32, 1, unroll=2)
        def _t(i):
            lane = lane0_in + i * 4
            for s in range(4):
                sidx = jnp.full((16,), 0, jnp.int32) + (lane + s)
                for c0 in (0, 16):
                    vals = plsc.load_gather(in_v, [iota16 + c0, sidx])
                    out_b[row0_out + i, pl.ds(s * 32 + c0, 16)] = vals

    for wt, tail, t4 in ((wt_sr, tail_sr, t4_sr), (wt_tg, tail_tg, t4_tg)):

        def grp_body(g, _, wt=wt, tail=tail, t4=t4):
            cg = wid * GRP_PER_W + g

            @pl.when(cg < N_GRP)
            def _full():
                pltpu.sync_copy(wt.at[:, pl.ds(cg * GRP * TCOL, GRP * TCOL)],
                                in_v)
                transpose_cols(GRP, 0, 0)
                pltpu.async_copy(out_b,
                                 t4.at[pl.ds(cg * GRP * 32, GRP * 32)],
                                 wsem).wait()

            @pl.when(cg == N_GRP)
            def _partial():
                base = N_GRP * GRP * TCOL
                pltpu.sync_copy(
                    wt.at[:, pl.ds(base, 4 * TCOL)],
                    in_v.at[:, pl.ds(0, 4 * TCOL)])
                transpose_cols(4, 0, 0)
                pltpu.sync_copy(tail, out_b.at[pl.ds(4 * 32, TAIL // 4)])
                pltpu.async_copy(
                    out_b.at[pl.ds(0, 4 * 32 + TAIL // 4)],
                    t4.at[pl.ds(cg * GRP * 32, 4 * 32 + TAIL // 4)],
                    wsem).wait()

            return 0

        lax.fori_loop(0, GRP_PER_W, grp_body, 0)


@functools.partial(
    pl.kernel,
    mesh=plsc.VectorSubcoreMesh(core_axis_name="c", subcore_axis_name="s"),
    out_type=(
        jax.ShapeDtypeStruct(OUT5, jnp.float32),
        jax.ShapeDtypeStruct(OUT5, jnp.float32),
    ),
    scratch_types=[
        pltpu.VMEM((CHUNK,), jnp.int32),
        pltpu.VMEM((CHUNK, EMBED_DIM), jnp.float32),
        pltpu.VMEM((NUM_COLS, EMBED_DIM // 8, 8, 128), jnp.float32),
        pltpu.SemaphoreType.DMA((2,)),
        pltpu.SemaphoreType.DMA,
        pltpu.SemaphoreType.DMA,
    ],
    compiler_params=pltpu.CompilerParams(
        use_tc_tiling_on_sc=False, needs_layout_passes=False),
)
def _double_gather(w_sr, w_tg, idx_sr, idx_tg, o_sr, o_tg,
                   idx_v, rows_v, out_v, isem, gsem, wsem):
    wid = lax.axis_index("s") * NC + lax.axis_index("c")
    base = wid * B_PER_W
    r0 = wid * N_RBLK
    iota16 = lax.iota(jnp.int32, 16)
    iota_r = iota16 * NUM_COLS

    for t, (w, idx, o) in enumerate(((w_sr, idx_sr, o_sr), (w_tg, idx_tg, o_tg))):

        def rblk_body(rb, _, w=w, idx=idx, o=o):
            for half in range(2):
                pltpu.async_copy(
                    idx.at[pl.ds(base + (2 * rb + half) * CHUNK, CHUNK)],
                    idx_v, isem.at[0]).wait()
                pltpu.async_copy(w.at[idx_v], rows_v, gsem).wait()

                @plsc.parallel_loop(0, NUM_COLS, 1, unroll=2)
                def _fill(col, half=half):
                    for h in range(EMBED_DIM // 8):
                        for l in range(8):
                            cidx = jnp.full((16,), h * 8 + l, jnp.int32)
                            for k in range(0, HALF, 16):
                                ridx = iota_r + (k * NUM_COLS + col)
                                vals = plsc.load_gather(rows_v, [ridx, cidx])
                                out_v[col, h, l,
                                      pl.ds(half * HALF + k, 16)] = vals

            def wb_body(col, _, o=o, rb=rb):
                for h in range(EMBED_DIM // 8):
                    pltpu.async_copy(out_v.at[col, h], o.at[col, h, r0 + rb],
                                     wsem)
                return 0

            lax.fori_loop(0, NUM_COLS, wb_body, 0)

            def wb_drain(col, _, o=o, rb=rb):
                for h in range(EMBED_DIM // 8):
                    pltpu.make_async_copy(out_v.at[col, h],
                                          o.at[col, h, r0 + rb], wsem).wait()
                return 0

            lax.fori_loop(0, NUM_COLS, wb_drain, 0)
            return 0

        lax.fori_loop(0, N_RBLK, rblk_body, 0)


def kernel(sr_data, tg_data, W_sr, W_tg):
    idx_sr = sr_data.reshape(B)
    idx_tg = tg_data.reshape(B)
    tail_sr = W_sr[N_TCOLS * TCOL:].reshape(TAIL // 4, 128)
    tail_tg = W_tg[N_TCOLS * TCOL:].reshape(TAIL // 4, 128)
    t4_sr, t4_tg = _repack(W_sr.T, W_tg.T, tail_sr, tail_tg)
    o_sr, o_tg = _double_gather(t4_sr.reshape(NUM_V, EMBED_DIM),
                                t4_tg.reshape(NUM_V, EMBED_DIM),
                                idx_sr, idx_tg)

    def unpack(o5):
        return o5.transpose(2, 4, 0, 1, 3).reshape(NUM_ROWS, NUM_COLS, EMBED_DIM)

    return (unpack(o_sr), unpack(o_tg))


# speedup vs baseline: 1.0921x; 1.0921x over previous
import functools

import jax
import jax.numpy as jnp
from jax import lax
from jax.experimental import pallas as pl
from jax.experimental.pallas import tpu as pltpu
from jax.experimental.pallas import tpu_sc as plsc

NUM_ROWS = 16384
NUM_COLS = 20
EMBED_DIM = 32
B = NUM_ROWS * NUM_COLS

NC = 2
NS = 16
NW = NC * NS
ROWS_PER_W = NUM_ROWS // NW
B_PER_W = B // NW
RBLK = 128
HALF = RBLK // 2
CHUNK = HALF * NUM_COLS
N_RBLK = ROWS_PER_W // RBLK

OUT5 = (NUM_COLS, EMBED_DIM // 8, NUM_ROWS // 128, 8, 128)

NUM_V = 1000000
TCOL = 128
N_TCOLS = NUM_V // TCOL
TAIL = NUM_V - N_TCOLS * TCOL
GRP = 6
N_GRP = N_TCOLS // GRP
GRP_PER_W = 42
GLANES = GRP * TCOL
GROWS = GRP * 32


@functools.partial(
    pl.kernel,
    mesh=plsc.VectorSubcoreMesh(core_axis_name="c", subcore_axis_name="s"),
    out_type=(
        jax.ShapeDtypeStruct((NUM_V // 4, 128), jnp.float32),
        jax.ShapeDtypeStruct((NUM_V // 4, 128), jnp.float32),
    ),
    scratch_types=[
        pltpu.VMEM((2, 32, GLANES), jnp.float32),
        pltpu.VMEM((GROWS, 128), jnp.float32),
        pltpu.SemaphoreType.DMA((2,)),
        pltpu.SemaphoreType.DMA,
    ],
    compiler_params=pltpu.CompilerParams(
        use_tc_tiling_on_sc=True, needs_layout_passes=False),
)
def _repack(wt_sr, wt_tg, tail_sr, tail_tg, t4_sr, t4_tg,
            in_v, out_b, ssem, wsem):
    wid = lax.axis_index("s") * NC + lax.axis_index("c")
    iota16 = lax.iota(jnp.int32, 16)

    def stage(cg, b, wt):
        pltpu.async_copy(
            wt.at[:, pl.ds(cg * GLANES, GLANES)], in_v.at[b], ssem.at[b])

    def stage_wait(b, wt):
        pltpu.make_async_copy(
            wt.at[:, pl.ds(0, GLANES)], in_v.at[b], ssem.at[b]).wait()

    def wb_wait(t4):
        pltpu.make_async_copy(out_b, t4.at[pl.ds(0, GROWS)], wsem).wait()

    for wt, tail, t4 in ((wt_sr, tail_sr, t4_sr), (wt_tg, tail_tg, t4_tg)):
        first = wid * GRP_PER_W

        @pl.when(first < N_GRP)
        def _prime(wt=wt):
            stage(first, 0, wt)

        def pair_body(p, _, wt=wt, tail=tail, t4=t4):
            for b in (0, 1):
                g = p * 2 + b
                cg = first + g

                @pl.when(cg < N_GRP)
                def _full(g=g, cg=cg, b=b, wt=wt, t4=t4):
                    @pl.when(cg + 1 < jnp.minimum(first + GRP_PER_W, N_GRP))
                    def _prefetch():
                        stage(cg + 1, 1 - b, wt)

                    stage_wait(b, wt)

                    @pl.when(g >= 1)
                    def _reuse_gate():
                        wb_wait(t4)

                    src = in_v.at[b]

                    @plsc.parallel_loop(0, GROWS, 1, unroll=2)
                    def _t(i):
                        for s in range(4):
                            sidx = jnp.full((16,), 0, jnp.int32) + (i * 4 + s)
                            for c0 in (0, 16):
                                vals = plsc.load_gather(
                                    src, [iota16 + c0, sidx])
                                out_b[i, pl.ds(s * 32 + c0, 16)] = vals

                    pltpu.async_copy(
                        out_b, t4.at[pl.ds(cg * GROWS, GROWS)], wsem)

                @pl.when(cg == N_GRP)
                def _tail(tail=tail, t4=t4):
                    pltpu.sync_copy(
                        tail, t4.at[pl.ds(N_GRP * GROWS, TAIL // 4)])

            return 0

        lax.fori_loop(0, GRP_PER_W // 2, pair_body, 0)

        @pl.when(first < N_GRP)
        def _drain(t4=t4):
            wb_wait(t4)


@functools.partial(
    pl.kernel,
    mesh=plsc.VectorSubcoreMesh(core_axis_name="c", subcore_axis_name="s"),
    out_type=(
        jax.ShapeDtypeStruct(OUT5, jnp.float32),
        jax.ShapeDtypeStruct(OUT5, jnp.float32),
    ),
    scratch_types=[
        pltpu.VMEM((CHUNK,), jnp.int32),
        pltpu.VMEM((CHUNK, EMBED_DIM), jnp.float32),
        pltpu.VMEM((NUM_COLS, EMBED_DIM // 8, 8, 128), jnp.float32),
        pltpu.SemaphoreType.DMA((2,)),
        pltpu.SemaphoreType.DMA,
        pltpu.SemaphoreType.DMA,
    ],
    compiler_params=pltpu.CompilerParams(
        use_tc_tiling_on_sc=False, needs_layout_passes=False),
)
def _double_gather(w_sr, w_tg, idx_sr, idx_tg, o_sr, o_tg,
                   idx_v, rows_v, out_v, isem, gsem, wsem):
    wid = lax.axis_index("s") * NC + lax.axis_index("c")
    base = wid * B_PER_W
    r0 = wid * N_RBLK
    iota16 = lax.iota(jnp.int32, 16)
    iota_r = iota16 * NUM_COLS

    for t, (w, idx, o) in enumerate(((w_sr, idx_sr, o_sr), (w_tg, idx_tg, o_tg))):

        def rblk_body(rb, _, w=w, idx=idx, o=o):
            for half in range(2):
                pltpu.async_copy(
                    idx.at[pl.ds(base + (2 * rb + half) * CHUNK, CHUNK)],
                    idx_v, isem.at[0]).wait()
                pltpu.async_copy(w.at[idx_v], rows_v, gsem).wait()

                @plsc.parallel_loop(0, NUM_COLS, 1, unroll=2)
                def _fill(col, half=half):
                    for h in range(EMBED_DIM // 8):
                        for l in range(8):
                            cidx = jnp.full((16,), h * 8 + l, jnp.int32)
                            for k in range(0, HALF, 16):
                                ridx = iota_r + (k * NUM_COLS + col)
                                vals = plsc.load_gather(rows_v, [ridx, cidx])
                                out_v[col, h, l,
                                      pl.ds(half * HALF + k, 16)] = vals

            def wb_body(col, _, o=o, rb=rb):
                for h in range(EMBED_DIM // 8):
                    pltpu.async_copy(out_v.at[col, h], o.at[col, h, r0 + rb],
                                     wsem)
                return 0

            lax.fori_loop(0, NUM_COLS, wb_body, 0)

            def wb_drain(col, _, o=o, rb=rb):
                for h in range(EMBED_DIM // 8):
                    pltpu.make_async_copy(out_v.at[col, h],
                                          o.at[col, h, r0 + rb], wsem).wait()
                return 0

            lax.fori_loop(0, NUM_COLS, wb_drain, 0)
            return 0

        lax.fori_loop(0, N_RBLK, rblk_body, 0)


def kernel(sr_data, tg_data, W_sr, W_tg):
    idx_sr = sr_data.reshape(B)
    idx_tg = tg_data.reshape(B)
    tail_sr = W_sr[N_TCOLS * TCOL:].reshape(TAIL // 4, 128)
    tail_tg = W_tg[N_TCOLS * TCOL:].reshape(TAIL // 4, 128)
    t4_sr, t4_tg = _repack(W_sr.T, W_tg.T, tail_sr, tail_tg)
    o_sr, o_tg = _double_gather(t4_sr.reshape(NUM_V, EMBED_DIM),
                                t4_tg.reshape(NUM_V, EMBED_DIM),
                                idx_sr, idx_tg)

    def unpack(o5):
        return o5.transpose(2, 4, 0, 1, 3).reshape(NUM_ROWS, NUM_COLS, EMBED_DIM)

    return (unpack(o_sr), unpack(o_tg))


# speedup vs baseline: 1.1373x; 1.0413x over previous
import functools

import jax
import jax.numpy as jnp
from jax import lax
from jax.experimental import pallas as pl
from jax.experimental.pallas import tpu as pltpu
from jax.experimental.pallas import tpu_sc as plsc

NUM_ROWS = 16384
NUM_COLS = 20
EMBED_DIM = 32
B = NUM_ROWS * NUM_COLS

NC = 2
NS = 16
NW = NC * NS
ROWS_PER_W = NUM_ROWS // NW
B_PER_W = B // NW
RBLK = 128
QTR = RBLK // 4
CHUNK = QTR * NUM_COLS
N_RBLK = ROWS_PER_W // RBLK

OUT5 = (NUM_COLS, EMBED_DIM // 8, NUM_ROWS // 128, 8, 128)


@functools.partial(
    pl.kernel,
    mesh=plsc.VectorSubcoreMesh(core_axis_name="c", subcore_axis_name="s"),
    out_type=(
        jax.ShapeDtypeStruct(OUT5, jnp.float32),
        jax.ShapeDtypeStruct(OUT5, jnp.float32),
    ),
    scratch_types=[
        pltpu.VMEM((2, CHUNK), jnp.int32),
        pltpu.VMEM((2, CHUNK, EMBED_DIM), jnp.float32),
        pltpu.VMEM((NUM_COLS, EMBED_DIM // 8, 8, 128), jnp.float32),
        pltpu.SemaphoreType.DMA((2,)),
        pltpu.SemaphoreType.DMA((2,)),
        pltpu.SemaphoreType.DMA,
    ],
    compiler_params=pltpu.CompilerParams(
        use_tc_tiling_on_sc=False, needs_layout_passes=False),
)
def _double_gather(w_sr, w_tg, idx_sr, idx_tg, o_sr, o_tg,
                   idx_v, rows_v, out_v, isem, gsem, wsem):
    wid = lax.axis_index("s") * NC + lax.axis_index("c")
    base = wid * B_PER_W
    r0 = wid * N_RBLK
    iota16 = lax.iota(jnp.int32, 16)
    iota_r = iota16 * NUM_COLS

    for t, (w, idx, o) in enumerate(((w_sr, idx_sr, o_sr), (w_tg, idx_tg, o_tg))):

        def rblk_body(rb, _, w=w, idx=idx, o=o):
            pltpu.async_copy(idx.at[pl.ds(base + 4 * rb * CHUNK, CHUNK)],
                             idx_v.at[0], isem.at[0]).wait()
            pltpu.async_copy(w.at[idx_v.at[0]], rows_v.at[0], gsem.at[0])
            for q in range(4):
                b = q % 2
                if q < 3:
                    nb = 1 - b
                    pltpu.async_copy(
                        idx.at[pl.ds(base + (4 * rb + q + 1) * CHUNK, CHUNK)],
                        idx_v.at[nb], isem.at[nb]).wait()
                    pltpu.async_copy(w.at[idx_v.at[nb]], rows_v.at[nb],
                                     gsem.at[nb])
                pltpu.make_async_copy(w.at[idx_v.at[b]], rows_v.at[b],
                                      gsem.at[b]).wait()

                @plsc.parallel_loop(0, NUM_COLS, 1, unroll=2)
                def _fill(col, q=q, b=b):
                    for h in range(EMBED_DIM // 8):
                        for l in range(8):
                            cidx = jnp.full((16,), h * 8 + l, jnp.int32)
                            for k in range(0, QTR, 16):
                                ridx = iota_r + (k * NUM_COLS + col)
                                vals = plsc.load_gather(rows_v.at[b],
                                                        [ridx, cidx])
                                out_v[col, h, l,
                                      pl.ds(q * QTR + k, 16)] = vals

            def wb_body(col, _, o=o, rb=rb):
                for h in range(EMBED_DIM // 8):
                    pltpu.async_copy(out_v.at[col, h], o.at[col, h, r0 + rb],
                                     wsem)
                return 0

            lax.fori_loop(0, NUM_COLS, wb_body, 0)

            def wb_drain(col, _, o=o, rb=rb):
                for h in range(EMBED_DIM // 8):
                    pltpu.make_async_copy(out_v.at[col, h],
                                          o.at[col, h, r0 + rb], wsem).wait()
                return 0

            lax.fori_loop(0, NUM_COLS, wb_drain, 0)
            return 0

        lax.fori_loop(0, N_RBLK, rblk_body, 0)


def kernel(sr_data, tg_data, W_sr, W_tg):
    idx_sr = sr_data.reshape(B)
    idx_tg = tg_data.reshape(B)
    o_sr, o_tg = _double_gather(W_sr, W_tg, idx_sr, idx_tg)

    def unpack(o5):
        return o5.transpose(2, 4, 0, 1, 3).reshape(NUM_ROWS, NUM_COLS, EMBED_DIM)

    return (unpack(o_sr), unpack(o_tg))
